# baseline (speedup 1.0000x reference)
import functools

import jax
import jax.numpy as jnp
from jax import lax
from jax.experimental import pallas as pl
from jax.experimental.pallas import tpu as pltpu
from jax.experimental.pallas import tpu_sc as plsc

D_MODEL = 768
N_TOKENS = 8192
NC, NS, L = 2, 16, 16
NW = NC * NS
PER_W = N_TOKENS // NW
CHUNK = 16
NCHUNK = PER_W // CHUNK
NBUF = 5
LEAD = 3
VECS = D_MODEL // L


def _emb_kernel(toks_hbm, pos_hbm, we_hbm, wpos_hbm, out_hbm,
                tok_idx, pos_idx,
                tb0, pb0, tb1, pb1, tb2, pb2, tb3, pb3, tb4, pb4,
                gs0, gs1, gs2, gs3, gs4, ws0, ws1, ws2, ws3, ws4):
    wid = lax.axis_index("s") * NC + lax.axis_index("c")
    base = wid * PER_W

    tokbufs = (tb0, tb1, tb2, tb3, tb4)
    posbufs = (pb0, pb1, pb2, pb3, pb4)
    gsems = (gs0, gs1, gs2, gs3, gs4)
    wsems = (ws0, ws1, ws2, ws3, ws4)

    pltpu.sync_copy(toks_hbm.at[wid], tok_idx)
    pltpu.sync_copy(pos_hbm.at[wid], pos_idx)

    def fire(g):
        slot = g % NBUF
        return (
            pltpu.async_copy(we_hbm.at[tok_idx.at[g]], tokbufs[slot], gsems[slot]),
            pltpu.async_copy(wpos_hbm.at[pos_idx.at[g]], posbufs[slot], gsems[slot]),
        )

    gh = [None] * NBUF
    wh = [None] * NBUF
    for g in range(LEAD):
        gh[g] = fire(g)

    for g in range(NCHUNK):
        cur = g % NBUF
        if g + LEAD < NCHUNK:
            slot = (g + LEAD) % NBUF
            if wh[slot] is not None:
                wh[slot].wait()
            gh[slot] = fire(g + LEAD)

        gh[cur][0].wait()
        gh[cur][1].wait()

        tb, pb = tokbufs[cur], posbufs[cur]

        def row_body(r, carry):
            for j in range(VECS):
                sl = pl.ds(j * L, L)
                tb[r, sl] = tb[r, sl] + pb[r, sl]
            return carry

        lax.fori_loop(0, CHUNK, row_body, 0)

        wh[cur] = pltpu.async_copy(
            tb, out_hbm.at[pl.ds(base + g * CHUNK, CHUNK)], wsems[cur])

    for slot in range(NBUF):
        if wh[slot] is not None:
            wh[slot].wait()


@jax.jit
def kernel(toks, pos, W_E, W_pos):
    B, S = toks.shape
    toks32 = toks.reshape(NW, NCHUNK, CHUNK).astype(jnp.int32)
    pos32 = pos.reshape(NW, NCHUNK, CHUNK).astype(jnp.int32)

    run = functools.partial(
        pl.kernel,
        out_type=jax.ShapeDtypeStruct((N_TOKENS, D_MODEL), jnp.float32),
        mesh=plsc.VectorSubcoreMesh(core_axis_name="c", subcore_axis_name="s"),
        scratch_types=(
            [pltpu.VMEM((NCHUNK, CHUNK), jnp.int32)] * 2
            + [pltpu.VMEM((CHUNK, D_MODEL), jnp.float32)] * (2 * NBUF)
            + [pltpu.SemaphoreType.DMA] * (2 * NBUF)
        ),
    )(_emb_kernel)
    flat = run(toks32, pos32, W_E, W_pos)
    return flat.reshape(B, S, D_MODEL)


# speedup vs baseline: 1.0535x; 1.0535x over previous
import functools

import jax
import jax.numpy as jnp
from jax import lax
from jax.experimental import pallas as pl
from jax.experimental.pallas import tpu as pltpu
from jax.experimental.pallas import tpu_sc as plsc

D_MODEL = 768
N_TOKENS = 8192
SEQ = 2048
NC, NS, L = 2, 16, 16
NW = NC * NS
PER_W = N_TOKENS // NW
W_PER_ROW = SEQ // PER_W
CHUNK = 32
NCHUNK = PER_W // CHUNK
VECS = D_MODEL // L


def _emb_kernel(toks_hbm, pos_hbm, we_hbm, wpos_hbm, out_hbm,
                tok_idx, pos_idx,
                tb0, pb0, tb1, pb1,
                gs0, gs1, ws0, ws1):
    wid = lax.axis_index("s") * NC + lax.axis_index("c")
    base = wid * PER_W
    brow = wid // W_PER_ROW
    bcol = (wid % W_PER_ROW) * PER_W

    tokbufs = (tb0, tb1)
    posbufs = (pb0, pb1)
    gsems = (gs0, gs1)
    wsems = (ws0, ws1)

    pltpu.sync_copy(toks_hbm.at[brow, pl.ds(bcol, PER_W)], tok_idx)
    pltpu.sync_copy(pos_hbm.at[brow, pl.ds(bcol, PER_W)], pos_idx)

    def fire(g, slot):
        tsl = tok_idx.at[pl.ds(g * CHUNK, CHUNK)]
        psl = pos_idx.at[pl.ds(g * CHUNK, CHUNK)]
        return (
            pltpu.async_copy(we_hbm.at[tsl], tokbufs[slot], gsems[slot]),
            pltpu.async_copy(wpos_hbm.at[psl], posbufs[slot], gsems[slot]),
        )

    gh = [None, None]
    wh = [None, None]
    gh[0] = fire(0, 0)
    for g in range(NCHUNK):
        cur = g % 2
        nxt = 1 - cur
        if g + 1 < NCHUNK:
            if wh[nxt] is not None:
                wh[nxt].wait()
            gh[nxt] = fire(g + 1, nxt)

        gh[cur][0].wait()
        gh[cur][1].wait()

        tb, pb = tokbufs[cur], posbufs[cur]

        def row_body(r, carry):
            for j in range(VECS):
                sl = pl.ds(j * L, L)
                tb[r, sl] = tb[r, sl] + pb[r, sl]
            return carry

        lax.fori_loop(0, CHUNK, row_body, 0)

        wh[cur] = pltpu.async_copy(
            tb, out_hbm.at[pl.ds(base + g * CHUNK, CHUNK)], wsems[cur])

    wh[0].wait()
    wh[1].wait()


@jax.jit
def kernel(toks, pos, W_E, W_pos):
    B, S = toks.shape
    toks32 = toks.astype(jnp.int32)
    pos32 = pos.astype(jnp.int32)

    run = functools.partial(
        pl.kernel,
        out_type=jax.ShapeDtypeStruct((N_TOKENS, D_MODEL), jnp.float32),
        mesh=plsc.VectorSubcoreMesh(core_axis_name="c", subcore_axis_name="s"),
        scratch_types=(
            [pltpu.VMEM((PER_W,), jnp.int32)] * 2
            + [pltpu.VMEM((CHUNK, D_MODEL), jnp.float32)] * 4
            + [pltpu.SemaphoreType.DMA] * 4
        ),
    )(_emb_kernel)
    flat = run(toks32, pos32, W_E, W_pos)
    return flat.reshape(B, S, D_MODEL)


# speedup vs baseline: 1.1319x; 1.0744x over previous
import functools

import jax
import jax.numpy as jnp
from jax import lax
from jax.experimental import pallas as pl
from jax.experimental.pallas import tpu as pltpu
from jax.experimental.pallas import tpu_sc as plsc

D_MODEL = 768
BATCH = 4
SEQ = 2048
N_TOKENS = BATCH * SEQ
NC, NS, L = 2, 16, 16
NW = NC * NS
PER_W = N_TOKENS // NW
W_PER_ROW = SEQ // PER_W
CHUNK = 32
NCHUNK = PER_W // CHUNK
VECS = D_MODEL // L


def _emb_kernel(toks_hbm, pos_hbm, we_hbm, wpos_hbm, out_hbm,
                tok_idx, pos_idx,
                tb0, pb0, tb1, pb1,
                gs0, gs1, ws0, ws1):
    wid = lax.axis_index("s") * NC + lax.axis_index("c")
    brow = wid // W_PER_ROW
    bcol = (wid % W_PER_ROW) * PER_W

    tokbufs = (tb0, tb1)
    posbufs = (pb0, pb1)
    gsems = (gs0, gs1)
    wsems = (ws0, ws1)

    h1 = pltpu.async_copy(toks_hbm.at[brow, pl.ds(bcol, PER_W)], tok_idx, gs0)
    h2 = pltpu.async_copy(pos_hbm.at[brow, pl.ds(bcol, PER_W)], pos_idx, gs1)
    h1.wait()
    h2.wait()

    def fire(c, slot):
        tsl = tok_idx.at[pl.ds(c * CHUNK, CHUNK)]
        psl = pos_idx.at[pl.ds(c * CHUNK, CHUNK)]
        pltpu.async_copy(we_hbm.at[tsl], tokbufs[slot], gsems[slot])
        pltpu.async_copy(wpos_hbm.at[psl], posbufs[slot], gsems[slot])

    def drain_gathers(slot):
        pltpu.make_async_copy(we_hbm.at[pl.ds(0, CHUNK)], tokbufs[slot],
                              gsems[slot]).wait()
        pltpu.make_async_copy(we_hbm.at[pl.ds(0, CHUNK)], posbufs[slot],
                              gsems[slot]).wait()

    def drain_wb(slot):
        pltpu.make_async_copy(tokbufs[slot],
                              out_hbm.at[0, pl.ds(0, CHUNK)],
                              wsems[slot]).wait()

    fire(0, 0)

    def pair_body(i, carry):
        for b in range(2):
            c = 2 * i + b
            cur = b
            nxt = 1 - b

            @pl.when(c + 1 < NCHUNK)
            def _():
                @pl.when(c >= 1)
                def _():
                    drain_wb(nxt)
                fire(c + 1, nxt)

            drain_gathers(cur)

            tb, pb = tokbufs[cur], posbufs[cur]

            def row_body(r, rc):
                for j in range(VECS):
                    sl = pl.ds(j * L, L)
                    plsc.addupdate(tb.at[r, sl], pb[r, sl])
                return rc

            lax.fori_loop(0, CHUNK, row_body, 0)

            pltpu.async_copy(
                tb, out_hbm.at[brow, pl.ds(bcol + c * CHUNK, CHUNK)],
                wsems[cur])
        return carry

    lax.fori_loop(0, NCHUNK // 2, pair_body, 0)

    drain_wb(0)
    drain_wb(1)


@jax.jit
def kernel(toks, pos, W_E, W_pos):
    B, S = toks.shape
    toks32 = toks.astype(jnp.int32)
    pos32 = pos.astype(jnp.int32)

    run = functools.partial(
        pl.kernel,
        out_type=jax.ShapeDtypeStruct((BATCH, SEQ, D_MODEL), jnp.float32),
        mesh=plsc.VectorSubcoreMesh(core_axis_name="c", subcore_axis_name="s"),
        scratch_types=(
            [pltpu.VMEM((PER_W,), jnp.int32)] * 2
            + [pltpu.VMEM((CHUNK, D_MODEL), jnp.float32)] * 4
            + [pltpu.SemaphoreType.DMA] * 4
        ),
    )(_emb_kernel)
    return run(toks32, pos32, W_E, W_pos)


# speedup vs baseline: 1.1377x; 1.0052x over previous
import functools

import jax
import jax.numpy as jnp
from jax import lax
from jax.experimental import pallas as pl
from jax.experimental.pallas import tpu as pltpu
from jax.experimental.pallas import tpu_sc as plsc

D_MODEL = 768
BATCH = 4
SEQ = 2048
N_TOKENS = BATCH * SEQ
NC, NS, L = 2, 16, 16
NW = NC * NS
PER_W = N_TOKENS // NW
W_PER_ROW = SEQ // PER_W
CHUNK = 16
NCHUNK = PER_W // CHUNK
NBUF = 4
LEAD = 3
VECS = D_MODEL // L


def _emb_kernel(toks_hbm, pos_hbm, we_hbm, wpos_hbm, out_hbm,
                tok_idx, pos_idx,
                tb0, pb0, tb1, pb1, tb2, pb2, tb3, pb3,
                gs0, gs1, gs2, gs3, ws0, ws1, ws2, ws3):
    wid = lax.axis_index("s") * NC + lax.axis_index("c")
    brow = wid // W_PER_ROW
    bcol = (wid % W_PER_ROW) * PER_W

    tokbufs = (tb0, tb1, tb2, tb3)
    posbufs = (pb0, pb1, pb2, pb3)
    gsems = (gs0, gs1, gs2, gs3)
    wsems = (ws0, ws1, ws2, ws3)

    h1 = pltpu.async_copy(toks_hbm.at[brow, pl.ds(bcol, PER_W)], tok_idx, gs0)
    h2 = pltpu.async_copy(pos_hbm.at[brow, pl.ds(bcol, PER_W)], pos_idx, gs1)
    h1.wait()
    h2.wait()

    def fire(c, slot):
        tsl = tok_idx.at[pl.ds(c * CHUNK, CHUNK)]
        psl = pos_idx.at[pl.ds(c * CHUNK, CHUNK)]
        pltpu.async_copy(we_hbm.at[tsl], tokbufs[slot], gsems[slot])
        pltpu.async_copy(wpos_hbm.at[psl], posbufs[slot], gsems[slot])

    def drain_gathers(slot):
        pltpu.make_async_copy(we_hbm.at[pl.ds(0, CHUNK)], tokbufs[slot],
                              gsems[slot]).wait()
        pltpu.make_async_copy(we_hbm.at[pl.ds(0, CHUNK)], posbufs[slot],
                              gsems[slot]).wait()

    def drain_wb(slot):
        pltpu.make_async_copy(tokbufs[slot],
                              out_hbm.at[0, pl.ds(0, CHUNK)],
                              wsems[slot]).wait()

    for c0 in range(LEAD):
        fire(c0, c0)

    def quad_body(i, carry):
        for b in range(NBUF):
            c = NBUF * i + b
            cur = b
            ahead = (b + LEAD) % NBUF

            @pl.when(c + LEAD < NCHUNK)
            def _():
                @pl.when(c >= 1)
                def _():
                    drain_wb(ahead)
                fire(c + LEAD, ahead)

            drain_gathers(cur)

            tb, pb = tokbufs[cur], posbufs[cur]

            def row_body(r, rc):
                for j in range(VECS):
                    sl = pl.ds(j * L, L)
                    plsc.addupdate(tb.at[r, sl], pb[r, sl])
                return rc

            lax.fori_loop(0, CHUNK, row_body, 0)

            pltpu.async_copy(
                tb, out_hbm.at[brow, pl.ds(bcol + c * CHUNK, CHUNK)],
                wsems[cur])
        return carry

    lax.fori_loop(0, NCHUNK // NBUF, quad_body, 0)

    for slot in range(NBUF):
        drain_wb(slot)


@jax.jit
def kernel(toks, pos, W_E, W_pos):
    B, S = toks.shape
    toks32 = toks.astype(jnp.int32)
    pos32 = pos.astype(jnp.int32)

    run = functools.partial(
        pl.kernel,
        out_type=jax.ShapeDtypeStruct((BATCH, SEQ, D_MODEL), jnp.float32),
        mesh=plsc.VectorSubcoreMesh(core_axis_name="c", subcore_axis_name="s"),
        scratch_types=(
            [pltpu.VMEM((PER_W,), jnp.int32)] * 2
            + [pltpu.VMEM((CHUNK, D_MODEL), jnp.float32)] * (2 * NBUF)
            + [pltpu.SemaphoreType.DMA] * (2 * NBUF)
        ),
    )(_emb_kernel)
    return run(toks32, pos32, W_E, W_pos)
